# Initial kernel scaffold; baseline (speedup 1.0000x reference)
import jax
import jax.numpy as jnp
from jax.experimental import pallas as pl


def kernel(inputs, edge_index, W1, attn_l1, attn_r1, W2, attn_l2, attn_r2):
    raise NotImplementedError("write your pallas kernel here")


# speedup vs baseline: 33.3494x; 33.3494x over previous
import functools

import jax
import jax.numpy as jnp
from jax import lax
from jax.experimental import pallas as pl
from jax.experimental.pallas import tpu as pltpu
from jax.experimental.pallas import tpu_sc as plsc

N = 10000
E = 320000
NP = 10240
EP = 327680
NC = 2
NS = 16
NW = NC * NS
EPW = EP // NW
ROWS_PER_TILE = NP // NS
CB = 128
NCH = EPW // CB
IR = EP // 128
IRP = IR + 8

_mesh = plsc.VectorSubcoreMesh(core_axis_name="c", subcore_axis_name="s")
_params = pltpu.CompilerParams(use_tc_tiling_on_sc=False)


def _splat(v, k):
    idx = jnp.zeros((16,), jnp.int32) + k
    dn = lax.GatherDimensionNumbers(
        offset_dims=(), collapsed_slice_dims=(0,), start_index_map=(0,))
    return lax.gather(v, idx[:, None], dn, (1,),
                      mode=lax.GatherScatterMode.PROMISE_IN_BOUNDS)


def _dup16(v):
    idx = (lax.iota(jnp.int32, 16) & 7) + 8
    dn = lax.GatherDimensionNumbers(
        offset_dims=(), collapsed_slice_dims=(0,), start_index_map=(0,))
    return lax.gather(v, idx[:, None], dn, (1,),
                      mode=lax.GatherScatterMode.PROMISE_IN_BOUNDS)


def _edge_sc(src2d, dst2d, srcT, dstT, zeros_acc, feat=None, row_w=16):
    with_feat = feat is not None
    nj = row_w // 16

    scratch = [
        pltpu.VMEM((1, 128), jnp.int32),
        pltpu.VMEM((1, 128), jnp.int32),
        pltpu.VMEM((1, 128), jnp.int32),
        pltpu.VMEM((1, 128), jnp.int32),
        pltpu.VMEM((CB, 16), jnp.float32),
        pltpu.VMEM((CB, 16), jnp.float32),
        pltpu.VMEM((CB, 16), jnp.float32),
        pltpu.VMEM((CB, 16), jnp.float32),
        pltpu.VMEM_SHARED((NP, row_w), jnp.float32),
        pltpu.SemaphoreType.DMA,
        pltpu.SemaphoreType.DMA,
        pltpu.SemaphoreType.DMA,
        pltpu.SemaphoreType.DMA,
    ]
    if with_feat:
        scratch += [
            pltpu.VMEM((CB, row_w), jnp.float32),
            pltpu.VMEM((CB, row_w), jnp.float32),
            pltpu.VMEM((CB, 16), jnp.float32),
        ]
    else:
        scratch += [pltpu.VMEM((CB, 16), jnp.float32)]

    def k(*refs):
        if with_feat:
            (src_h, dst_h, srcT_h, dstT_h, z_h, feat_h, acc_h,
             sidx0, sidx1, didx0, didx1, sr0, sr1, dr0, dr1, acc,
             sg0, sg1, si0, si1, fb0, fb1, alphab) = refs
            featbs = (fb0, fb1)
        else:
            (src_h, dst_h, srcT_h, dstT_h, z_h, acc_h,
             sidx0, sidx1, didx0, didx1, sr0, sr1, dr0, dr1, acc,
             sg0, sg1, si0, si1, outb) = refs
        sidx = (sidx0, sidx1)
        didx = (didx0, didx1)
        srows = (sr0, sr1)
        drows = (dr0, dr1)
        sg = (sg0, sg1)
        si = (si0, si1)

        cid = lax.axis_index("c")
        sid = lax.axis_index("s")
        wid = cid * NS + sid
        irow0 = wid * NCH

        r0 = sid * ROWS_PER_TILE
        pltpu.sync_copy(z_h.at[pl.ds(r0, ROWS_PER_TILE)],
                        acc.at[pl.ds(r0, ROWS_PER_TILE)])
        plsc.subcore_barrier()

        def issue_idx(c, p):
            pltpu.async_copy(src_h.at[pl.ds(irow0 + c, 1)], sidx[p], si[p])
            pltpu.async_copy(dst_h.at[pl.ds(irow0 + c, 1)], didx[p], si[p])

        def wait_idx(p):
            pltpu.make_async_copy(src_h.at[pl.ds(irow0, 1)], sidx[p],
                                  si[p]).wait()
            pltpu.make_async_copy(dst_h.at[pl.ds(irow0, 1)], didx[p],
                                  si[p]).wait()

        def issue_gathers(p):
            pltpu.async_copy(srcT_h.at[sidx[p].at[0]], srows[p], sg[p])
            pltpu.async_copy(dstT_h.at[didx[p].at[0]], drows[p], sg[p])
            if with_feat:
                pltpu.async_copy(feat_h.at[sidx[p].at[0]], featbs[p], sg[p])

        def wait_gathers(p):
            pltpu.make_async_copy(srcT_h.at[sidx[p].at[0]], srows[p],
                                  sg[p]).wait()
            pltpu.make_async_copy(dstT_h.at[didx[p].at[0]], drows[p],
                                  sg[p]).wait()
            if with_feat:
                pltpu.make_async_copy(feat_h.at[sidx[p].at[0]], featbs[p],
                                      sg[p]).wait()

        def compute(p):
            if with_feat:
                fb = featbs[p]

                def row(r, carry):
                    v1 = srows[p][r, :]
                    v2 = drows[p][r, :]
                    e = v1 + v2
                    e = jnp.where(e >= 0.0, e, 0.2 * e)
                    alphab[r, :] = jnp.exp(e) * _dup16(v2)
                    return carry
                lax.fori_loop(0, CB, row, 0, unroll=4)

                def mrow(r, carry):
                    av = alphab[r, :]
                    for j in range(nj):
                        hcol = j if row_w == 128 else 0
                        aj = _splat(av, hcol)
                        fb[r, pl.ds(j * 16, 16)] = fb[r, pl.ds(j * 16, 16)] * aj
                    return carry
                lax.fori_loop(0, CB, mrow, 0, unroll=2)
            else:
                def row(r, carry):
                    e = srows[p][r, :] + drows[p][r, :]
                    e = jnp.where(e >= 0.0, e, 0.2 * e)
                    outb[r, :] = jnp.exp(e)
                    return carry
                lax.fori_loop(0, CB, row, 0, unroll=4)

        def scatter(p):
            src_buf = featbs[p] if with_feat else outb
            pltpu.sync_copy(src_buf, acc.at[didx[p].at[0]], add=True)

        pltpu.sync_copy(src_h.at[pl.ds(irow0, 1)], sidx[0])
        pltpu.sync_copy(dst_h.at[pl.ds(irow0, 1)], didx[0])
        pltpu.sync_copy(src_h.at[pl.ds(irow0 + 1, 1)], sidx[1])
        pltpu.sync_copy(dst_h.at[pl.ds(irow0 + 1, 1)], didx[1])
        issue_gathers(0)
        wait_gathers(0)
        issue_gathers(1)
        compute(0)
        scatter(0)
        issue_idx(2, 0)

        def body(i, carry):
            c1 = 2 * i + 1
            wait_gathers(1)
            wait_idx(0)
            issue_gathers(0)
            compute(1)
            scatter(1)
            issue_idx(c1 + 2, 1)

            wait_gathers(0)
            wait_idx(1)
            issue_gathers(1)
            compute(0)
            scatter(0)
            issue_idx(c1 + 3, 0)
            return carry
        lax.fori_loop(0, (NCH - 2) // 2, body, 0)

        wait_gathers(1)
        compute(1)
        scatter(1)
        wait_idx(0)

        plsc.subcore_barrier()
        pltpu.sync_copy(acc.at[pl.ds(r0, ROWS_PER_TILE)],
                        acc_h.at[cid, pl.ds(r0, ROWS_PER_TILE)])

    kern = functools.partial(
        pl.kernel,
        out_type=jax.ShapeDtypeStruct((NC, NP, row_w), jnp.float32),
        mesh=_mesh,
        compiler_params=_params,
        scratch_types=scratch,
    )(k)
    if with_feat:
        return kern(src2d, dst2d, srcT, dstT, zeros_acc, feat)
    return kern(src2d, dst2d, srcT, dstT, zeros_acc)


def _dense(x, W, Asrc16, Adst16):
    n_blk = NP // 1024
    row_w = W.shape[1]

    def body(x_ref, w_ref, as_ref, ad_ref, feat_ref, s_ref, d_ref):
        feat = jnp.dot(x_ref[...], w_ref[...],
                       preferred_element_type=jnp.float32)
        feat_ref[...] = feat
        s_ref[...] = jnp.dot(feat, as_ref[...],
                             preferred_element_type=jnp.float32)
        d_ref[...] = jnp.dot(feat, ad_ref[...],
                             preferred_element_type=jnp.float32)

    return pl.pallas_call(
        body,
        grid=(n_blk,),
        in_specs=[
            pl.BlockSpec((1024, x.shape[1]), lambda i: (i, 0)),
            pl.BlockSpec((x.shape[1], row_w), lambda i: (0, 0)),
            pl.BlockSpec((row_w, 16), lambda i: (0, 0)),
            pl.BlockSpec((row_w, 16), lambda i: (0, 0)),
        ],
        out_specs=[
            pl.BlockSpec((1024, row_w), lambda i: (i, 0)),
            pl.BlockSpec((1024, 16), lambda i: (i, 0)),
            pl.BlockSpec((1024, 16), lambda i: (i, 0)),
        ],
        out_shape=[
            jax.ShapeDtypeStruct((NP, row_w), jnp.float32),
            jax.ShapeDtypeStruct((NP, 16), jnp.float32),
            jax.ShapeDtypeStruct((NP, 16), jnp.float32),
        ],
    )(x, W, Asrc16, Adst16)


def _dense2(p0, p1, W, Asrc16, Adst16):
    n_blk = NP // 1024
    row_w = W.shape[1]

    def body(p0_ref, p1_ref, w_ref, as_ref, ad_ref, feat_ref, s_ref, d_ref):
        h = jax.nn.relu(p0_ref[...] + p1_ref[...])
        feat = jnp.dot(h, w_ref[...], preferred_element_type=jnp.float32)
        feat_ref[...] = feat
        s_ref[...] = jnp.dot(feat, as_ref[...],
                             preferred_element_type=jnp.float32)
        d_ref[...] = jnp.dot(feat, ad_ref[...],
                             preferred_element_type=jnp.float32)

    return pl.pallas_call(
        body,
        grid=(n_blk,),
        in_specs=[
            pl.BlockSpec((1024, 128), lambda i: (i, 0)),
            pl.BlockSpec((1024, 128), lambda i: (i, 0)),
            pl.BlockSpec((128, row_w), lambda i: (0, 0)),
            pl.BlockSpec((row_w, 16), lambda i: (0, 0)),
            pl.BlockSpec((row_w, 16), lambda i: (0, 0)),
        ],
        out_specs=[
            pl.BlockSpec((1024, row_w), lambda i: (i, 0)),
            pl.BlockSpec((1024, 16), lambda i: (i, 0)),
            pl.BlockSpec((1024, 16), lambda i: (i, 0)),
        ],
        out_shape=[
            jax.ShapeDtypeStruct((NP, row_w), jnp.float32),
            jax.ShapeDtypeStruct((NP, 16), jnp.float32),
            jax.ShapeDtypeStruct((NP, 16), jnp.float32),
        ],
    )(p0, p1, W, Asrc16, Adst16)


def _comb(sp0, sp1, dstTa):
    n_blk = NP // 1024

    def body(a_ref, b_ref, t_ref, o_ref):
        s = a_ref[...] + b_ref[...]
        rs = 1.0 / jnp.maximum(s, 1e-30)
        o_ref[...] = jnp.concatenate([t_ref[:, :8], rs[:, :8]], axis=1)

    return pl.pallas_call(
        body,
        grid=(n_blk,),
        in_specs=[pl.BlockSpec((1024, 16), lambda i: (i, 0))] * 3,
        out_specs=pl.BlockSpec((1024, 16), lambda i: (i, 0)),
        out_shape=jax.ShapeDtypeStruct((NP, 16), jnp.float32),
    )(sp0, sp1, dstTa)


def _final(q0, q1):
    n_blk = NP // 1024

    def body(a_ref, b_ref, o_ref):
        o_ref[...] = a_ref[:, :40] + b_ref[:, :40]

    return pl.pallas_call(
        body,
        grid=(n_blk,),
        in_specs=[pl.BlockSpec((1024, 48), lambda i: (i, 0))] * 2,
        out_specs=pl.BlockSpec((1024, 40), lambda i: (i, 0)),
        out_shape=jax.ShapeDtypeStruct((NP, 40), jnp.float32),
    )(q0, q1)


def kernel(inputs, edge_index, W1, attn_l1, attn_r1, W2, attn_l2, attn_r2):
    x = jnp.pad(inputs, ((0, NP - N), (0, 0)))
    pad = jnp.full((IRP * 128 - E,), N, jnp.int32)
    src2d = jnp.concatenate([edge_index[0], pad]).reshape(IRP, 128)
    dst2d = jnp.concatenate([edge_index[1], pad]).reshape(IRP, 128)

    eye = jnp.eye(8, dtype=jnp.float32)
    Al = (eye[:, None, :] * attn_l1[0][:, :, None]).reshape(128, 8)
    Ar = (eye[:, None, :] * attn_r1[0][:, :, None]).reshape(128, 8)
    Asrc1 = jnp.tile(Al, (1, 2))
    Adst1 = jnp.tile(Ar, (1, 2))
    W2p = jnp.pad(W2, ((0, 0), (0, 8)))
    A2l = jnp.tile(jnp.pad(attn_l2.reshape(40, 1), ((0, 8), (0, 0))), (1, 16))
    A2r = jnp.tile(jnp.pad(attn_r2.reshape(40, 1), ((0, 8), (0, 0))), (1, 16))

    z16 = jnp.zeros((NP, 16), jnp.float32)
    z48 = jnp.zeros((NP, 48), jnp.float32)
    z128 = jnp.zeros((NP, 128), jnp.float32)

    feat1, srcT1, dstTa1 = _dense(x, W1, Asrc1, Adst1)
    spart1 = _edge_sc(src2d, dst2d, srcT1, dstTa1, z16)
    dstTb1 = _comb(spart1[0], spart1[1], dstTa1)
    out1p = _edge_sc(src2d, dst2d, srcT1, dstTb1, z128, feat=feat1,
                     row_w=128)

    feat2, srcT2, dstTa2 = _dense2(out1p[0], out1p[1], W2p, A2l, A2r)
    spart2 = _edge_sc(src2d, dst2d, srcT2, dstTa2, z16)
    dstTb2 = _comb(spart2[0], spart2[1], dstTa2)
    out2p = _edge_sc(src2d, dst2d, srcT2, dstTb2, z48, feat=feat2, row_w=48)

    logits = _final(out2p[0], out2p[1])
    return logits[:N]


# speedup vs baseline: 35.3414x; 1.0597x over previous
import functools

import jax
import jax.numpy as jnp
from jax import lax
from jax.experimental import pallas as pl
from jax.experimental.pallas import tpu as pltpu
from jax.experimental.pallas import tpu_sc as plsc

N = 10000
E = 320000
NP = 10240
EP = 327680
NC = 2
NS = 16
NW = NC * NS
EPW = EP // NW
ROWS_PER_TILE = NP // NS
IR = EP // 128
IRP = IR + 8

_mesh = plsc.VectorSubcoreMesh(core_axis_name="c", subcore_axis_name="s")
_params = pltpu.CompilerParams(use_tc_tiling_on_sc=False)


def _splat(v, k):
    idx = jnp.zeros((16,), jnp.int32) + k
    dn = lax.GatherDimensionNumbers(
        offset_dims=(), collapsed_slice_dims=(0,), start_index_map=(0,))
    return lax.gather(v, idx[:, None], dn, (1,),
                      mode=lax.GatherScatterMode.PROMISE_IN_BOUNDS)


def _dup16(v):
    idx = (lax.iota(jnp.int32, 16) & 7) + 8
    dn = lax.GatherDimensionNumbers(
        offset_dims=(), collapsed_slice_dims=(0,), start_index_map=(0,))
    return lax.gather(v, idx[:, None], dn, (1,),
                      mode=lax.GatherScatterMode.PROMISE_IN_BOUNDS)


def _edge_sc(src2d, dst2d, srcT, dstT, zeros_acc, feat=None, row_w=16,
             CB=256):
    with_feat = feat is not None
    nj = row_w // 16
    KB = CB // 128
    NCH = EPW // CB

    scratch = [
        pltpu.VMEM((KB, 128), jnp.int32),
        pltpu.VMEM((KB, 128), jnp.int32),
        pltpu.VMEM((KB, 128), jnp.int32),
        pltpu.VMEM((KB, 128), jnp.int32),
        pltpu.VMEM((CB, 16), jnp.float32),
        pltpu.VMEM((CB, 16), jnp.float32),
        pltpu.VMEM((CB, 16), jnp.float32),
        pltpu.VMEM((CB, 16), jnp.float32),
        pltpu.VMEM_SHARED((NP, row_w), jnp.float32),
        pltpu.SemaphoreType.DMA,
        pltpu.SemaphoreType.DMA,
        pltpu.SemaphoreType.DMA,
        pltpu.SemaphoreType.DMA,
    ]
    if with_feat:
        scratch += [
            pltpu.VMEM((CB, row_w), jnp.float32),
            pltpu.VMEM((CB, row_w), jnp.float32),
        ]
    else:
        scratch += [pltpu.VMEM((CB, 16), jnp.float32)]

    def k(*refs):
        if with_feat:
            (src_h, dst_h, srcT_h, dstT_h, z_h, feat_h, acc_h,
             sidx0, sidx1, didx0, didx1, sr0, sr1, dr0, dr1, acc,
             sg0, sg1, si0, si1, fb0, fb1) = refs
            featbs = (fb0, fb1)
        else:
            (src_h, dst_h, srcT_h, dstT_h, z_h, acc_h,
             sidx0, sidx1, didx0, didx1, sr0, sr1, dr0, dr1, acc,
             sg0, sg1, si0, si1, outb) = refs
        sidx = (sidx0, sidx1)
        didx = (didx0, didx1)
        srows = (sr0, sr1)
        drows = (dr0, dr1)
        sg = (sg0, sg1)
        si = (si0, si1)

        cid = lax.axis_index("c")
        sid = lax.axis_index("s")
        wid = cid * NS + sid
        irow0 = wid * NCH * KB

        r0 = sid * ROWS_PER_TILE
        pltpu.sync_copy(z_h.at[pl.ds(r0, ROWS_PER_TILE)],
                        acc.at[pl.ds(r0, ROWS_PER_TILE)])
        plsc.subcore_barrier()

        def issue_idx(c, p):
            pltpu.async_copy(src_h.at[pl.ds(irow0 + c * KB, KB)],
                             sidx[p], si[p])
            pltpu.async_copy(dst_h.at[pl.ds(irow0 + c * KB, KB)],
                             didx[p], si[p])

        def wait_idx(p):
            pltpu.make_async_copy(src_h.at[pl.ds(irow0, KB)], sidx[p],
                                  si[p]).wait()
            pltpu.make_async_copy(dst_h.at[pl.ds(irow0, KB)], didx[p],
                                  si[p]).wait()

        def issue_gathers(p):
            for j in range(KB):
                sl = pl.ds(j * 128, 128)
                pltpu.async_copy(srcT_h.at[sidx[p].at[j]],
                                 srows[p].at[sl], sg[p])
                pltpu.async_copy(dstT_h.at[didx[p].at[j]],
                                 drows[p].at[sl], sg[p])
                if with_feat:
                    pltpu.async_copy(feat_h.at[sidx[p].at[j]],
                                     featbs[p].at[sl], sg[p])

        def wait_gathers(p):
            for j in range(KB):
                sl = pl.ds(j * 128, 128)
                pltpu.make_async_copy(srcT_h.at[sidx[p].at[j]],
                                      srows[p].at[sl], sg[p]).wait()
                pltpu.make_async_copy(dstT_h.at[didx[p].at[j]],
                                      drows[p].at[sl], sg[p]).wait()
                if with_feat:
                    pltpu.make_async_copy(feat_h.at[sidx[p].at[j]],
                                          featbs[p].at[sl], sg[p]).wait()

        def compute(p):
            if with_feat:
                fb = featbs[p]

                def row(r, carry):
                    v1 = srows[p][r, :]
                    v2 = drows[p][r, :]
                    e = v1 + v2
                    e = jnp.where(e >= 0.0, e, 0.2 * e)
                    av = jnp.exp(e) * _dup16(v2)
                    for j in range(nj):
                        hcol = j if row_w == 128 else 0
                        aj = _splat(av, hcol)
                        fb[r, pl.ds(j * 16, 16)] = fb[r, pl.ds(j * 16, 16)] * aj
                    return carry
                lax.fori_loop(0, CB, row, 0, unroll=4)
            else:
                def row(r, carry):
                    e = srows[p][r, :] + drows[p][r, :]
                    e = jnp.where(e >= 0.0, e, 0.2 * e)
                    outb[r, :] = jnp.exp(e)
                    return carry
                lax.fori_loop(0, CB, row, 0, unroll=8)

        def scatter(p):
            src_buf = featbs[p] if with_feat else outb
            for j in range(KB):
                sl = pl.ds(j * 128, 128)
                pltpu.sync_copy(src_buf.at[sl], acc.at[didx[p].at[j]],
                                add=True)

        pltpu.sync_copy(src_h.at[pl.ds(irow0, KB)], sidx[0])
        pltpu.sync_copy(dst_h.at[pl.ds(irow0, KB)], didx[0])
        pltpu.sync_copy(src_h.at[pl.ds(irow0 + KB, KB)], sidx[1])
        pltpu.sync_copy(dst_h.at[pl.ds(irow0 + KB, KB)], didx[1])
        issue_gathers(0)
        wait_gathers(0)
        issue_gathers(1)
        compute(0)
        scatter(0)
        issue_idx(2, 0)

        def body(i, carry):
            c1 = 2 * i + 1
            wait_gathers(1)
            wait_idx(0)
            issue_gathers(0)
            compute(1)
            scatter(1)
            issue_idx(c1 + 2, 1)

            wait_gathers(0)
            wait_idx(1)
            issue_gathers(1)
            compute(0)
            scatter(0)
            issue_idx(c1 + 3, 0)
            return carry
        lax.fori_loop(0, (NCH - 2) // 2, body, 0)

        wait_gathers(1)
        compute(1)
        scatter(1)
        wait_idx(0)

        plsc.subcore_barrier()
        pltpu.sync_copy(acc.at[pl.ds(r0, ROWS_PER_TILE)],
                        acc_h.at[cid, pl.ds(r0, ROWS_PER_TILE)])

    kern = functools.partial(
        pl.kernel,
        out_type=jax.ShapeDtypeStruct((NC, NP, row_w), jnp.float32),
        mesh=_mesh,
        compiler_params=_params,
        scratch_types=scratch,
    )(k)
    if with_feat:
        return kern(src2d, dst2d, srcT, dstT, zeros_acc, feat)
    return kern(src2d, dst2d, srcT, dstT, zeros_acc)


def _dense(x, W, Asrc16, Adst16):
    n_blk = NP // 1024
    row_w = W.shape[1]

    def body(x_ref, w_ref, as_ref, ad_ref, feat_ref, s_ref, d_ref):
        feat = jnp.dot(x_ref[...], w_ref[...],
                       preferred_element_type=jnp.float32)
        feat_ref[...] = feat
        s_ref[...] = jnp.dot(feat, as_ref[...],
                             preferred_element_type=jnp.float32)
        d_ref[...] = jnp.dot(feat, ad_ref[...],
                             preferred_element_type=jnp.float32)

    return pl.pallas_call(
        body,
        grid=(n_blk,),
        in_specs=[
            pl.BlockSpec((1024, x.shape[1]), lambda i: (i, 0)),
            pl.BlockSpec((x.shape[1], row_w), lambda i: (0, 0)),
            pl.BlockSpec((row_w, 16), lambda i: (0, 0)),
            pl.BlockSpec((row_w, 16), lambda i: (0, 0)),
        ],
        out_specs=[
            pl.BlockSpec((1024, row_w), lambda i: (i, 0)),
            pl.BlockSpec((1024, 16), lambda i: (i, 0)),
            pl.BlockSpec((1024, 16), lambda i: (i, 0)),
        ],
        out_shape=[
            jax.ShapeDtypeStruct((NP, row_w), jnp.float32),
            jax.ShapeDtypeStruct((NP, 16), jnp.float32),
            jax.ShapeDtypeStruct((NP, 16), jnp.float32),
        ],
    )(x, W, Asrc16, Adst16)


def _dense2(p0, p1, W, Asrc16, Adst16):
    n_blk = NP // 1024
    row_w = W.shape[1]

    def body(p0_ref, p1_ref, w_ref, as_ref, ad_ref, feat_ref, s_ref, d_ref):
        h = jax.nn.relu(p0_ref[...] + p1_ref[...])
        feat = jnp.dot(h, w_ref[...], preferred_element_type=jnp.float32)
        feat_ref[...] = feat
        s_ref[...] = jnp.dot(feat, as_ref[...],
                             preferred_element_type=jnp.float32)
        d_ref[...] = jnp.dot(feat, ad_ref[...],
                             preferred_element_type=jnp.float32)

    return pl.pallas_call(
        body,
        grid=(n_blk,),
        in_specs=[
            pl.BlockSpec((1024, 128), lambda i: (i, 0)),
            pl.BlockSpec((1024, 128), lambda i: (i, 0)),
            pl.BlockSpec((128, row_w), lambda i: (0, 0)),
            pl.BlockSpec((row_w, 16), lambda i: (0, 0)),
            pl.BlockSpec((row_w, 16), lambda i: (0, 0)),
        ],
        out_specs=[
            pl.BlockSpec((1024, row_w), lambda i: (i, 0)),
            pl.BlockSpec((1024, 16), lambda i: (i, 0)),
            pl.BlockSpec((1024, 16), lambda i: (i, 0)),
        ],
        out_shape=[
            jax.ShapeDtypeStruct((NP, row_w), jnp.float32),
            jax.ShapeDtypeStruct((NP, 16), jnp.float32),
            jax.ShapeDtypeStruct((NP, 16), jnp.float32),
        ],
    )(p0, p1, W, Asrc16, Adst16)


def _comb(sp0, sp1, dstTa):
    n_blk = NP // 1024

    def body(a_ref, b_ref, t_ref, o_ref):
        s = a_ref[...] + b_ref[...]
        rs = 1.0 / jnp.maximum(s, 1e-30)
        o_ref[...] = jnp.concatenate([t_ref[:, :8], rs[:, :8]], axis=1)

    return pl.pallas_call(
        body,
        grid=(n_blk,),
        in_specs=[pl.BlockSpec((1024, 16), lambda i: (i, 0))] * 3,
        out_specs=pl.BlockSpec((1024, 16), lambda i: (i, 0)),
        out_shape=jax.ShapeDtypeStruct((NP, 16), jnp.float32),
    )(sp0, sp1, dstTa)


def _final(q0, q1):
    n_blk = NP // 1024

    def body(a_ref, b_ref, o_ref):
        o_ref[...] = a_ref[:, :40] + b_ref[:, :40]

    return pl.pallas_call(
        body,
        grid=(n_blk,),
        in_specs=[pl.BlockSpec((1024, 48), lambda i: (i, 0))] * 2,
        out_specs=pl.BlockSpec((1024, 40), lambda i: (i, 0)),
        out_shape=jax.ShapeDtypeStruct((NP, 40), jnp.float32),
    )(q0, q1)


def kernel(inputs, edge_index, W1, attn_l1, attn_r1, W2, attn_l2, attn_r2):
    x = jnp.pad(inputs, ((0, NP - N), (0, 0)))
    pad = jnp.full((IRP * 128 - E,), N, jnp.int32)
    src2d = jnp.concatenate([edge_index[0], pad]).reshape(IRP, 128)
    dst2d = jnp.concatenate([edge_index[1], pad]).reshape(IRP, 128)

    eye = jnp.eye(8, dtype=jnp.float32)
    Al = (eye[:, None, :] * attn_l1[0][:, :, None]).reshape(128, 8)
    Ar = (eye[:, None, :] * attn_r1[0][:, :, None]).reshape(128, 8)
    Asrc1 = jnp.tile(Al, (1, 2))
    Adst1 = jnp.tile(Ar, (1, 2))
    W2p = jnp.pad(W2, ((0, 0), (0, 8)))
    A2l = jnp.tile(jnp.pad(attn_l2.reshape(40, 1), ((0, 8), (0, 0))), (1, 16))
    A2r = jnp.tile(jnp.pad(attn_r2.reshape(40, 1), ((0, 8), (0, 0))), (1, 16))

    z16 = jnp.zeros((NP, 16), jnp.float32)
    z48 = jnp.zeros((NP, 48), jnp.float32)
    z128 = jnp.zeros((NP, 128), jnp.float32)

    feat1, srcT1, dstTa1 = _dense(x, W1, Asrc1, Adst1)
    spart1 = _edge_sc(src2d, dst2d, srcT1, dstTa1, z16, CB=512)
    dstTb1 = _comb(spart1[0], spart1[1], dstTa1)
    out1p = _edge_sc(src2d, dst2d, srcT1, dstTb1, z128, feat=feat1,
                     row_w=128, CB=128)

    feat2, srcT2, dstTa2 = _dense2(out1p[0], out1p[1], W2p, A2l, A2r)
    spart2 = _edge_sc(src2d, dst2d, srcT2, dstTa2, z16, CB=512)
    dstTb2 = _comb(spart2[0], spart2[1], dstTa2)
    out2p = _edge_sc(src2d, dst2d, srcT2, dstTb2, z48, feat=feat2,
                     row_w=48, CB=128)

    logits = _final(out2p[0], out2p[1])
    return logits[:N]
